# Initial kernel scaffold; baseline (speedup 1.0000x reference)
import jax
import jax.numpy as jnp
from jax.experimental import pallas as pl


def kernel(x, edge_index, batch_index, W1, b1, W2, b2, Wout, bout):
    raise NotImplementedError("write your pallas kernel here")


# speedup vs baseline: 34.0367x; 34.0367x over previous
import functools

import jax
import jax.numpy as jnp
from jax import lax
from jax.experimental import pallas as pl
from jax.experimental.pallas import tpu as pltpu
from jax.experimental.pallas import tpu_sc as plsc

NC = 2
NS = 16
NW = NC * NS

N = 10000
NP = 10240
RPT = NP // NS
E = 320000
EW = E // NW

H = 128
G = 64

SCH, SK = 5, 2000
RCH, RK = 80, 125

_mesh = lambda: plsc.VectorSubcoreMesh(core_axis_name="c", subcore_axis_name="s")
_f32 = jnp.float32


def _wid():
    c = lax.axis_index("c")
    s = lax.axis_index("s")
    return c, s, s * NC + c


def _fill_zeros_1d(ref, n):
    def body(i, _):
        ref[pl.ds(i * 16, 16)] = jnp.zeros((16,), _f32)
        return 0
    lax.fori_loop(0, n // 16, body, 0)


def _deg_body(dst_hbm, deg0_hbm, deg1_hbm, dst_v, ones_v, zer_v, hist_s):
    c, s, w = _wid()

    def fill1(i, _):
        ones_v[pl.ds(i * 16, 16)] = jnp.ones((16,), _f32)
        return 0
    lax.fori_loop(0, SK // 16, fill1, 0)
    _fill_zeros_1d(zer_v, RPT)

    pltpu.sync_copy(zer_v, hist_s.at[pl.ds(s * RPT, RPT)])
    pltpu.sync_copy(dst_hbm.at[w], dst_v)
    plsc.subcore_barrier()

    for j in range(SCH):
        pltpu.sync_copy(ones_v, hist_s.at[dst_v.at[j, 0]], add=True)
    plsc.subcore_barrier()

    @pl.when(c == 0)
    def _():
        pltpu.sync_copy(hist_s.at[pl.ds(s * RPT, RPT)],
                        deg0_hbm.at[pl.ds(s * RPT, RPT)])

    @pl.when(c == 1)
    def _():
        pltpu.sync_copy(hist_s.at[pl.ds(s * RPT, RPT)],
                        deg1_hbm.at[pl.ds(s * RPT, RPT)])


@jax.jit
def _deg_call(dst3):
    return pl.kernel(
        _deg_body,
        out_type=[jax.ShapeDtypeStruct((NP,), _f32),
                  jax.ShapeDtypeStruct((NP,), _f32)],
        mesh=_mesh(),
        scratch_types=[
            pltpu.VMEM((SCH, 1, SK), jnp.int32),
            pltpu.VMEM((SK,), _f32),
            pltpu.VMEM((RPT,), _f32),
            pltpu.VMEM_SHARED((NP,), _f32),
        ],
    )(dst3)


def _agg1_body(xp_hbm, src_hbm, dst_hbm, t0_hbm, t1_hbm,
               xbuf_v, src_v, dst_v, val_v, zer_v, xp_s, acc_s, sem):
    c, s, w = _wid()
    _fill_zeros_1d(zer_v, RPT)
    pltpu.sync_copy(zer_v, acc_s.at[pl.ds(s * RPT, RPT)])
    pltpu.sync_copy(xp_hbm.at[pl.ds(s * RPT, RPT)], xbuf_v)
    pltpu.sync_copy(xbuf_v, xp_s.at[pl.ds(s * RPT, RPT)])
    pltpu.sync_copy(src_hbm.at[w], src_v)
    pltpu.sync_copy(dst_hbm.at[w], dst_v)
    plsc.subcore_barrier()

    for j in range(SCH):
        pltpu.async_copy(xp_s.at[src_v.at[j, 0]], val_v, sem).wait()
        pltpu.sync_copy(val_v, acc_s.at[dst_v.at[j, 0]], add=True)
    plsc.subcore_barrier()

    @pl.when(c == 0)
    def _():
        pltpu.sync_copy(acc_s.at[pl.ds(s * RPT, RPT)],
                        t0_hbm.at[pl.ds(s * RPT, RPT)])

    @pl.when(c == 1)
    def _():
        pltpu.sync_copy(acc_s.at[pl.ds(s * RPT, RPT)],
                        t1_hbm.at[pl.ds(s * RPT, RPT)])


@jax.jit
def _agg1_call(xp, src3, dst3):
    return pl.kernel(
        _agg1_body,
        out_type=[jax.ShapeDtypeStruct((NP,), _f32),
                  jax.ShapeDtypeStruct((NP,), _f32)],
        mesh=_mesh(),
        scratch_types=[
            pltpu.VMEM((RPT,), _f32),
            pltpu.VMEM((SCH, 1, SK), jnp.int32),
            pltpu.VMEM((SCH, 1, SK), jnp.int32),
            pltpu.VMEM((SK,), _f32),
            pltpu.VMEM((RPT,), _f32),
            pltpu.VMEM_SHARED((NP,), _f32),
            pltpu.VMEM_SHARED((NP,), _f32),
            pltpu.SemaphoreType.DMA,
        ],
    )(xp, src3, dst3)


def _agg2_body(mp_hbm, src_hbm, dst_hbm, a0_hbm, a1_hbm,
               src_v, dst_v, rows_v, zrow_v, acc_s, sem):
    c, s, w = _wid()

    def fz(i, _):
        r = i // 8
        l = (i % 8) * 16
        zrow_v[r, pl.ds(l, 16)] = jnp.zeros((16,), _f32)
        return 0
    lax.fori_loop(0, 64 * 8, fz, 0)

    for jj in range(RPT // 64):
        pltpu.sync_copy(zrow_v, acc_s.at[pl.ds(s * RPT + jj * 64, 64)])
    pltpu.sync_copy(src_hbm.at[w], src_v)
    pltpu.sync_copy(dst_hbm.at[w], dst_v)
    plsc.subcore_barrier()

    for j in range(RCH):
        pltpu.async_copy(mp_hbm.at[src_v.at[j, 0]], rows_v, sem).wait()
        pltpu.sync_copy(rows_v, acc_s.at[dst_v.at[j, 0]], add=True)
    plsc.subcore_barrier()

    @pl.when(c == 0)
    def _():
        for jj in range(RPT // 64):
            pltpu.sync_copy(acc_s.at[pl.ds(s * RPT + jj * 64, 64)],
                            a0_hbm.at[pl.ds(s * RPT + jj * 64, 64)])

    @pl.when(c == 1)
    def _():
        for jj in range(RPT // 64):
            pltpu.sync_copy(acc_s.at[pl.ds(s * RPT + jj * 64, 64)],
                            a1_hbm.at[pl.ds(s * RPT + jj * 64, 64)])


@jax.jit
def _agg2_call(mp, srcr, dstr):
    return pl.kernel(
        _agg2_body,
        out_type=[jax.ShapeDtypeStruct((NP, H), _f32),
                  jax.ShapeDtypeStruct((NP, H), _f32)],
        mesh=_mesh(),
        scratch_types=[
            pltpu.VMEM((RCH, 1, RK), jnp.int32),
            pltpu.VMEM((RCH, 1, RK), jnp.int32),
            pltpu.VMEM((RK, H), _f32),
            pltpu.VMEM((64, H), _f32),
            pltpu.VMEM_SHARED((NP, H), _f32),
            pltpu.SemaphoreType.DMA,
        ],
    )(mp, srcr, dstr)


def _tc1_body(d0_ref, d1_ref, x_ref, dis_ref, xp_ref):
    deg = 1.0 + d0_ref[...] + d1_ref[...]
    dis = lax.rsqrt(deg)
    dis_ref[...] = dis
    xp_ref[...] = dis * x_ref[...]


@jax.jit
def _tc1_call(deg0, deg1, xpad):
    return pl.pallas_call(
        _tc1_body,
        out_shape=[jax.ShapeDtypeStruct((NP,), _f32),
                   jax.ShapeDtypeStruct((NP,), _f32)],
    )(deg0, deg1, xpad)


_TC2B = 512


def _tc2_body(t0, t1, dis, xp, w1, b1, W2, mp_ref):
    sv = dis[...] * (xp[...] + t0[...] + t1[...])
    h1 = jnp.maximum(sv[:, None] * w1[...][None, :] + b1[...][None, :], 0.0)
    m = jnp.dot(h1, W2[...], preferred_element_type=_f32)
    mp_ref[...] = dis[...][:, None] * m


@jax.jit
def _tc2_call(t0, t1, dis, xp, w1, b1, W2):
    vec = pl.BlockSpec((_TC2B,), lambda i: (i,))
    return pl.pallas_call(
        _tc2_body,
        grid=(NP // _TC2B,),
        in_specs=[vec, vec, vec, vec,
                  pl.BlockSpec((H,), lambda i: (0,)),
                  pl.BlockSpec((H,), lambda i: (0,)),
                  pl.BlockSpec((H, H), lambda i: (0, 0))],
        out_specs=pl.BlockSpec((_TC2B, H), lambda i: (i, 0)),
        out_shape=jax.ShapeDtypeStruct((NP, H), _f32),
    )(t0, t1, dis, xp, w1, b1, W2)


_TC3B = 400


def _tc3_body(a0, a1, mp, dis, b2, bidx, wout, bout, out_ref,
              gmax_s, gsum_s, gcnt_s):
    i = pl.program_id(0)

    @pl.when(i == 0)
    def _():
        gmax_s[...] = jnp.full((G, H), -jnp.inf, _f32)
        gsum_s[...] = jnp.zeros((G, H), _f32)
        gcnt_s[...] = jnp.zeros((G, 128), _f32)

    h2 = jnp.maximum(
        dis[...] * (mp[...] + a0[...] + a1[...]) + b2[...][None, :],
        0.0)
    b = bidx[...]
    lo = b[0, 0]
    hi = b[_TC3B - 1, 0]

    def seg(g, _):
        mask = b == g
        msel = jnp.where(mask, h2, -jnp.inf)
        gmax_s[g, :] = jnp.maximum(gmax_s[g, :], jnp.max(msel, axis=0))
        ssel = jnp.where(mask, h2, 0.0)
        gsum_s[g, :] = gsum_s[g, :] + jnp.sum(ssel, axis=0)
        gcnt_s[g, :] = gcnt_s[g, :] + jnp.sum(mask.astype(_f32))
        return 0

    lax.fori_loop(lo, hi + 1, seg, 0)

    @pl.when(i == pl.num_programs(0) - 1)
    def _():
        gmean = gsum_s[...] / jnp.maximum(gcnt_s[...][:, :H], 1.0)
        wo = wout[...]
        logits = (jnp.dot(gmax_s[...], wo[:H, :], preferred_element_type=_f32)
                  + jnp.dot(gmean, wo[H:, :], preferred_element_type=_f32)
                  + bout[...][None, :])
        mx = jnp.max(logits, axis=1, keepdims=True)
        e = jnp.exp(logits - mx)
        out_ref[...] = e / jnp.sum(e, axis=1, keepdims=True)


@jax.jit
def _tc3_call(a0, a1, mp, dis2, b2, bidx2, Wout, bout):
    row = pl.BlockSpec((_TC3B, H), lambda i: (i, 0))
    return pl.pallas_call(
        _tc3_body,
        grid=(N // _TC3B,),
        in_specs=[row, row, row,
                  pl.BlockSpec((_TC3B, 1), lambda i: (i, 0)),
                  pl.BlockSpec((H,), lambda i: (0,)),
                  pl.BlockSpec((_TC3B, 1), lambda i: (i, 0)),
                  pl.BlockSpec((2 * H, 10), lambda i: (0, 0)),
                  pl.BlockSpec((10,), lambda i: (0,))],
        out_specs=pl.BlockSpec((G, 10), lambda i: (0, 0)),
        out_shape=jax.ShapeDtypeStruct((G, 10), _f32),
        scratch_shapes=[pltpu.VMEM((G, H), _f32),
                        pltpu.VMEM((G, H), _f32),
                        pltpu.VMEM((G, 128), _f32)],
    )(a0, a1, mp, dis2, b2, bidx2, Wout, bout)


def kernel(x, edge_index, batch_index, W1, b1, W2, b2, Wout, bout):
    src = edge_index[0]
    dst = edge_index[1]
    src3 = src.reshape(NW, SCH, 1, SK)
    dst3 = dst.reshape(NW, SCH, 1, SK)
    srcr = src.reshape(NW, RCH, 1, RK)
    dstr = dst.reshape(NW, RCH, 1, RK)
    xpad = jnp.pad(x[:, 0], (0, NP - N))
    bidx2 = batch_index.reshape(N, 1)

    deg0, deg1 = _deg_call(dst3)
    dis, xp = _tc1_call(deg0, deg1, xpad)
    t0, t1 = _agg1_call(xp, src3, dst3)
    mp = _tc2_call(t0, t1, dis, xp, W1.reshape(H), b1, W2)
    a0, a1 = _agg2_call(mp, srcr, dstr)
    return _tc3_call(a0, a1, mp, dis.reshape(NP, 1), b2, bidx2, Wout, bout)
